# baseline (speedup 1.0000x reference)
import jax
import jax.numpy as jnp
from jax import lax
from jax.experimental import pallas as pl
from jax.experimental.pallas import tpu as pltpu
from jax.experimental.pallas import tpu_sc as plsc

N = 10000
E = 320000
D_IN = 128
D_HID = 128
D_OUT = 64
DF = 64

NUM_SC = 2
NUM_TILES = 16
NW = NUM_SC * NUM_TILES
LANES = 16

CHUNK = 128
KSUB = 1
NBUF = 2
GLEAD = 1
SLAG = 1
SUPER = CHUNK * KSUB
EDGES_PER_TILE = -(-E // (NW * SUPER * NBUF)) * SUPER * NBUF
E_PAD = EDGES_PER_TILE * NW
NCHUNKS = EDGES_PER_TILE // SUPER
N_PAD = 10112
ROWS_PER_TILE = N_PAD // NUM_TILES


def _make_segsum(with_cnt):
  mesh = plsc.VectorSubcoreMesh(core_axis_name="c", subcore_axis_name="s")

  out_type = [jax.ShapeDtypeStruct((NUM_SC, N_PAD, DF), jnp.float32)]
  if with_cnt:
    out_type.append(jax.ShapeDtypeStruct((NUM_SC, N_PAD), jnp.float32))

  nct = EDGES_PER_TILE // CHUNK

  scratch = [
      pltpu.VMEM((nct, CHUNK), jnp.int32),
      pltpu.VMEM((nct, CHUNK), jnp.int32),
      pltpu.VMEM((NBUF, CHUNK, DF), jnp.float32),
      pltpu.VMEM((ROWS_PER_TILE, DF), jnp.float32),
      pltpu.VMEM_SHARED((N_PAD, DF), jnp.float32),
      [pltpu.SemaphoreType.DMA] * NBUF,
      [pltpu.SemaphoreType.DMA] * NBUF,
  ]
  if with_cnt:
    scratch += [
        pltpu.VMEM((CHUNK,), jnp.float32),
        pltpu.VMEM((ROWS_PER_TILE,), jnp.float32),
        pltpu.VMEM_SHARED((N_PAD,), jnp.float32),
        pltpu.SemaphoreType.DMA,
    ]

  def body(table, src, dst, *refs):
    if with_cnt:
      (out, cnt_out, srcv, dstv, rows, zbuf, acc, gsem, ssem,
       ones, zbuf1, cntacc, csem) = refs
    else:
      out, srcv, dstv, rows, zbuf, acc, gsem, ssem = refs
    cid = lax.axis_index("c")
    sid = lax.axis_index("s")
    wid = sid * NUM_SC + cid
    tile_row0 = wid * nct
    row0 = sid * ROWS_PER_TILE

    def fire_gather(b, i):
      pltpu.async_copy(table.at[srcv.at[i]], rows.at[b], gsem[b])

    def drain_gather(b, i):
      pltpu.make_async_copy(table.at[srcv.at[i]], rows.at[b], gsem[b]).wait()

    def fire_scatter(b, i):
      pltpu.async_copy(rows.at[b], acc.at[dstv.at[i]], ssem[b], add=True)
      if with_cnt:
        pltpu.async_copy(ones, cntacc.at[dstv.at[i]], csem, add=True)

    def drain_scatter(b, i):
      pltpu.make_async_copy(rows.at[b], acc.at[dstv.at[i]], ssem[b]).wait()

    zvec = jnp.zeros((LANES,), jnp.float32)

    def zrow(r, _):
      for j in range(DF // LANES):
        zbuf[r, pl.ds(j * LANES, LANES)] = zvec
      return 0
    lax.fori_loop(0, ROWS_PER_TILE, zrow, 0)
    pltpu.sync_copy(zbuf, acc.at[pl.ds(row0, ROWS_PER_TILE)])
    if with_cnt:
      def zrow1(r, _):
        zbuf1[pl.ds(r * LANES, LANES)] = zvec
        return 0
      lax.fori_loop(0, ROWS_PER_TILE // LANES, zrow1, 0)
      pltpu.sync_copy(zbuf1, cntacc.at[pl.ds(row0, ROWS_PER_TILE)])
      onev = jnp.ones((LANES,), jnp.float32)
      for j in range(CHUNK // LANES):
        ones[pl.ds(j * LANES, LANES)] = onev

    pltpu.sync_copy(src.at[pl.ds(tile_row0, nct)], srcv)
    pltpu.sync_copy(dst.at[pl.ds(tile_row0, nct)], dstv)
    for v in range(GLEAD):
      fire_gather(v % NBUF, v)

    plsc.subcore_barrier()

    def outer(g, _):
      for b in range(NBUF):
        v = g * NBUF + b

        @pl.when(v >= SLAG)
        def _():
          drain_scatter((b - SLAG) % NBUF, v - SLAG)

        @pl.when(v + GLEAD < nct)
        def _():
          fire_gather((b + GLEAD) % NBUF, v + GLEAD)
        drain_gather(b, v)
        fire_scatter(b, v)
      return 0
    lax.fori_loop(0, nct // NBUF, outer, 0)

    for v in range(nct - SLAG, nct):
      drain_scatter(v % NBUF, v)
    if with_cnt:
      def cdrain(i, _):
        pltpu.make_async_copy(ones, cntacc.at[dstv.at[i]], csem).wait()
        return 0
      lax.fori_loop(0, nct, cdrain, 0)

    plsc.subcore_barrier()

    pltpu.sync_copy(acc.at[pl.ds(row0, ROWS_PER_TILE)],
                    out.at[cid, pl.ds(row0, ROWS_PER_TILE)])
    if with_cnt:
      pltpu.sync_copy(cntacc.at[pl.ds(row0, ROWS_PER_TILE)],
                      cnt_out.at[cid, pl.ds(row0, ROWS_PER_TILE)])

  return pl.kernel(
      body, out_type=out_type, mesh=mesh, scratch_types=scratch,
      compiler_params=pltpu.CompilerParams(use_tc_tiling_on_sc=False,
                                           needs_layout_passes=False),
      name=f"segsum_cnt{int(with_cnt)}")


_segsum_cnt = _make_segsum(with_cnt=True)
_segsum = _make_segsum(with_cnt=False)


def _mid_body(pa_ref, pb_ref, cnt_ref, x_ref, wl0_ref, wr0_ref, bias0_ref,
              wl1_ref, wr1_ref, bias1_ref, hl_ref, hr_ref):
  sa = pa_ref[0] + pa_ref[1]
  sb = pb_ref[0] + pb_ref[1]
  s = jnp.concatenate([sa, sb], axis=1)
  cnt = jnp.maximum(cnt_ref[0] + cnt_ref[1], 1.0)
  aggr = s / cnt
  x = x_ref[...]
  lin = (lax.dot_general(aggr, wl0_ref[...], (((1,), (1,)), ((), ())),
                         preferred_element_type=jnp.float32)
         + lax.dot_general(x, wr0_ref[...], (((1,), (1,)), ((), ())),
                           preferred_element_type=jnp.float32))
  h = jnp.maximum(lin + bias0_ref[...], 0.0)
  hl_ref[...] = lax.dot_general(h, wl1_ref[...], (((1,), (1,)), ((), ())),
                                preferred_element_type=jnp.float32)
  hr_ref[...] = (lax.dot_general(h, wr1_ref[...], (((1,), (1,)), ((), ())),
                                 preferred_element_type=jnp.float32)
                 + bias1_ref[...])


def _loss_body(p_ref, cnt_ref, hr_ref, y_ref, m_ref, out_ref):
  s = p_ref[0] + p_ref[1]
  cnt = jnp.maximum(cnt_ref[0] + cnt_ref[1], 1.0)
  logits = s / cnt + hr_ref[...]
  mx = jnp.max(logits, axis=1, keepdims=True)
  lse = mx + jnp.log(jnp.sum(jnp.exp(logits - mx), axis=1, keepdims=True))
  logp = logits - lse
  cols = lax.broadcasted_iota(jnp.int32, (N_PAD, D_OUT), 1)
  onehot = cols == y_ref[...]
  nll = -jnp.sum(jnp.where(onehot, logp, 0.0), axis=1, keepdims=True)
  m = m_ref[...]
  num = jnp.sum(nll * m)
  den = jnp.maximum(jnp.sum(m), 1.0)
  out_ref[0, 0] = num / den


def kernel(x_chunks, adj_chunks, y_chunks, train_mask_chunks,
           W_l0, b_l0, W_r0, b_r0, W_l1, b_l1, W_r1, b_r1):
  f32 = jnp.float32
  x_pad = jnp.zeros((N_PAD, D_IN), f32).at[:N].set(x_chunks)
  pad_e = E_PAD - E
  src = jnp.concatenate([adj_chunks[0], jnp.full((pad_e,), N, jnp.int32)])
  src = src.reshape(E_PAD // CHUNK, CHUNK)
  dst = jnp.concatenate([adj_chunks[1], jnp.full((pad_e,), N, jnp.int32)])
  dst = dst.reshape(E_PAD // CHUNK, CHUNK)
  y_pad = jnp.zeros((N_PAD, 1), jnp.int32).at[:N, 0].set(y_chunks)
  m_pad = jnp.zeros((N_PAD, 1), f32).at[:N, 0].set(
      train_mask_chunks.astype(f32))
  bias0 = (b_l0 + b_r0)[None, :]
  bias1 = (b_l1 + b_r1)[None, :]

  xa = x_pad[:, :DF]
  xb = x_pad[:, DF:]

  pa, cnt = _segsum_cnt(xa, src, dst)
  (pb,) = _segsum(xb, src, dst)
  cnt3 = cnt[:, :, None]

  hl, hr = pl.pallas_call(
      _mid_body,
      out_shape=[jax.ShapeDtypeStruct((N_PAD, D_OUT), f32),
                 jax.ShapeDtypeStruct((N_PAD, D_OUT), f32)],
  )(pa, pb, cnt3, x_pad, W_l0, W_r0, bias0, W_l1, W_r1, bias1)

  (p1,) = _segsum(hl, src, dst)

  loss = pl.pallas_call(
      _loss_body,
      out_shape=jax.ShapeDtypeStruct((1, 1), f32),
      out_specs=pl.BlockSpec(memory_space=pltpu.SMEM),
  )(p1, cnt3, hr, y_pad, m_pad)

  return loss.reshape(1)


# speedup vs baseline: 1.7692x; 1.7692x over previous
import jax
import jax.numpy as jnp
from jax import lax
from jax.experimental import pallas as pl
from jax.experimental.pallas import tpu as pltpu
from jax.experimental.pallas import tpu_sc as plsc

N = 10000
E = 320000
D_IN = 128
D_HID = 128
D_OUT = 64
DF = 32

NUM_SC = 2
NUM_TILES = 16
NW = NUM_SC * NUM_TILES
LANES = 16

CHUNK = 128
NBUF = 2
GLEAD = 1
SLAG = 1
EDGES_PER_TILE = -(-E // (NW * CHUNK * NBUF)) * CHUNK * NBUF
E_PAD = EDGES_PER_TILE * NW
N_PAD = 10112
ROWS_PER_TILE = N_PAD // NUM_TILES


def _make_segsum(npass, with_cnt):
  mesh = plsc.VectorSubcoreMesh(core_axis_name="c", subcore_axis_name="s")

  out_type = [jax.ShapeDtypeStruct((npass, NUM_SC, N_PAD, DF), jnp.float32)]
  if with_cnt:
    out_type.append(jax.ShapeDtypeStruct((NUM_SC, N_PAD), jnp.float32))

  nct = EDGES_PER_TILE // CHUNK

  scratch = [
      pltpu.VMEM((nct, CHUNK), jnp.int32),
      pltpu.VMEM((nct, CHUNK), jnp.int32),
      pltpu.VMEM((NBUF, CHUNK, DF), jnp.float32),
      pltpu.VMEM((ROWS_PER_TILE, DF), jnp.float32),
      pltpu.VMEM_SHARED((N_PAD, DF), jnp.float32),
      pltpu.VMEM_SHARED((N_PAD, DF), jnp.float32),
      [pltpu.SemaphoreType.DMA] * NBUF,
      [pltpu.SemaphoreType.DMA] * NBUF,
  ]
  if with_cnt:
    scratch += [
        pltpu.VMEM((CHUNK,), jnp.float32),
        pltpu.VMEM((ROWS_PER_TILE,), jnp.float32),
        pltpu.VMEM_SHARED((N_PAD,), jnp.float32),
        pltpu.SemaphoreType.DMA,
    ]

  def body(table, src, dst, *refs):
    if with_cnt:
      (out, cnt_out, srcv, dstv, rows, zbuf, stab, acc, gsem, ssem,
       ones, zbuf1, cntacc, csem) = refs
    else:
      out, srcv, dstv, rows, zbuf, stab, acc, gsem, ssem = refs
    cid = lax.axis_index("c")
    sid = lax.axis_index("s")
    wid = sid * NUM_SC + cid
    tile_row0 = wid * nct
    row0 = sid * ROWS_PER_TILE

    def fire_gather(b, i):
      pltpu.async_copy(stab.at[srcv.at[i]], rows.at[b], gsem[b])

    def drain_gather(b, i):
      pltpu.make_async_copy(stab.at[srcv.at[i]], rows.at[b], gsem[b]).wait()

    def fire_scatter(b, i, j):
      pltpu.async_copy(rows.at[b], acc.at[dstv.at[i]], ssem[b], add=True)
      if with_cnt:
        @pl.when(j == 0)
        def _():
          pltpu.async_copy(ones, cntacc.at[dstv.at[i]], csem, add=True)

    def drain_scatter(b, i):
      pltpu.make_async_copy(rows.at[b], acc.at[dstv.at[i]], ssem[b]).wait()

    zvec = jnp.zeros((LANES,), jnp.float32)

    def zrow(r, _):
      for j in range(DF // LANES):
        zbuf[r, pl.ds(j * LANES, LANES)] = zvec
      return 0
    lax.fori_loop(0, ROWS_PER_TILE, zrow, 0)
    if with_cnt:
      def zrow1(r, _):
        zbuf1[pl.ds(r * LANES, LANES)] = zvec
        return 0
      lax.fori_loop(0, ROWS_PER_TILE // LANES, zrow1, 0)
      onev = jnp.ones((LANES,), jnp.float32)
      for j in range(CHUNK // LANES):
        ones[pl.ds(j * LANES, LANES)] = onev
    pltpu.sync_copy(src.at[pl.ds(tile_row0, nct)], srcv)
    pltpu.sync_copy(dst.at[pl.ds(tile_row0, nct)], dstv)

    def one_pass(j, _):
      pltpu.sync_copy(table.at[j, pl.ds(row0, ROWS_PER_TILE)],
                      stab.at[pl.ds(row0, ROWS_PER_TILE)])
      pltpu.sync_copy(zbuf, acc.at[pl.ds(row0, ROWS_PER_TILE)])
      if with_cnt:
        @pl.when(j == 0)
        def _():
          pltpu.sync_copy(zbuf1, cntacc.at[pl.ds(row0, ROWS_PER_TILE)])

      plsc.subcore_barrier()

      for v in range(GLEAD):
        fire_gather(v % NBUF, v)

      def outer(g, _):
        for b in range(NBUF):
          v = g * NBUF + b

          @pl.when(v >= SLAG)
          def _():
            drain_scatter((b - SLAG) % NBUF, v - SLAG)

          @pl.when(v + GLEAD < nct)
          def _():
            fire_gather((b + GLEAD) % NBUF, v + GLEAD)
          drain_gather(b, v)
          fire_scatter(b, v, j)
        return 0
      lax.fori_loop(0, nct // NBUF, outer, 0)

      for v in range(nct - SLAG, nct):
        drain_scatter(v % NBUF, v)
      if with_cnt:
        @pl.when(j == 0)
        def _():
          def cdrain(i, _):
            pltpu.make_async_copy(ones, cntacc.at[dstv.at[i]], csem).wait()
            return 0
          lax.fori_loop(0, nct, cdrain, 0)

      plsc.subcore_barrier()

      pltpu.sync_copy(acc.at[pl.ds(row0, ROWS_PER_TILE)],
                      out.at[j, cid, pl.ds(row0, ROWS_PER_TILE)])
      if with_cnt:
        @pl.when(j == 0)
        def _():
          pltpu.sync_copy(cntacc.at[pl.ds(row0, ROWS_PER_TILE)],
                          cnt_out.at[cid, pl.ds(row0, ROWS_PER_TILE)])
      return 0

    lax.fori_loop(0, npass, one_pass, 0)

  return pl.kernel(
      body, out_type=out_type, mesh=mesh, scratch_types=scratch,
      compiler_params=pltpu.CompilerParams(use_tc_tiling_on_sc=False,
                                           needs_layout_passes=False),
      name=f"segsum{npass}_cnt{int(with_cnt)}")


_segsum4_cnt = _make_segsum(4, with_cnt=True)
_segsum2 = _make_segsum(2, with_cnt=False)


def _mid_body(p_ref, cnt_ref, x_ref, wl0_ref, wr0_ref, bias0_ref,
              wl1_ref, wr1_ref, bias1_ref, hl_ref, hr_ref):
  s = p_ref[0] + p_ref[1]
  cnt = jnp.maximum(cnt_ref[0] + cnt_ref[1], 1.0)
  aggr = s / cnt
  x = x_ref[...]
  lin = (lax.dot_general(aggr, wl0_ref[...], (((1,), (1,)), ((), ())),
                         preferred_element_type=jnp.float32)
         + lax.dot_general(x, wr0_ref[...], (((1,), (1,)), ((), ())),
                           preferred_element_type=jnp.float32))
  h = jnp.maximum(lin + bias0_ref[...], 0.0)
  hl_ref[...] = lax.dot_general(h, wl1_ref[...], (((1,), (1,)), ((), ())),
                                preferred_element_type=jnp.float32)
  hr_ref[...] = (lax.dot_general(h, wr1_ref[...], (((1,), (1,)), ((), ())),
                                 preferred_element_type=jnp.float32)
                 + bias1_ref[...])


def _loss_body(p_ref, cnt_ref, hr_ref, y_ref, m_ref, out_ref):
  s = p_ref[0] + p_ref[1]
  cnt = jnp.maximum(cnt_ref[0] + cnt_ref[1], 1.0)
  logits = s / cnt + hr_ref[...]
  mx = jnp.max(logits, axis=1, keepdims=True)
  lse = mx + jnp.log(jnp.sum(jnp.exp(logits - mx), axis=1, keepdims=True))
  logp = logits - lse
  cols = lax.broadcasted_iota(jnp.int32, (N_PAD, D_OUT), 1)
  onehot = cols == y_ref[...]
  nll = -jnp.sum(jnp.where(onehot, logp, 0.0), axis=1, keepdims=True)
  m = m_ref[...]
  num = jnp.sum(nll * m)
  den = jnp.maximum(jnp.sum(m), 1.0)
  out_ref[0, 0] = num / den


def kernel(x_chunks, adj_chunks, y_chunks, train_mask_chunks,
           W_l0, b_l0, W_r0, b_r0, W_l1, b_l1, W_r1, b_r1):
  f32 = jnp.float32
  x_pad = jnp.zeros((N_PAD, D_IN), f32).at[:N].set(x_chunks)
  pad_e = E_PAD - E
  src = jnp.concatenate([adj_chunks[0], jnp.full((pad_e,), N, jnp.int32)])
  src = src.reshape(E_PAD // CHUNK, CHUNK)
  dst = jnp.concatenate([adj_chunks[1], jnp.full((pad_e,), N, jnp.int32)])
  dst = dst.reshape(E_PAD // CHUNK, CHUNK)
  y_pad = jnp.zeros((N_PAD, 1), jnp.int32).at[:N, 0].set(y_chunks)
  m_pad = jnp.zeros((N_PAD, 1), f32).at[:N, 0].set(
      train_mask_chunks.astype(f32))
  bias0 = (b_l0 + b_r0)[None, :]
  bias1 = (b_l1 + b_r1)[None, :]

  x4 = x_pad.reshape(N_PAD, 4, DF).transpose(1, 0, 2)

  p0, cnt = _segsum4_cnt(x4, src, dst)
  p0c = p0.transpose(1, 2, 0, 3).reshape(NUM_SC, N_PAD, D_IN)
  cnt3 = cnt[:, :, None]

  hl, hr = pl.pallas_call(
      _mid_body,
      out_shape=[jax.ShapeDtypeStruct((N_PAD, D_OUT), f32),
                 jax.ShapeDtypeStruct((N_PAD, D_OUT), f32)],
  )(p0c, cnt3, x_pad, W_l0, W_r0, bias0, W_l1, W_r1, bias1)

  hl2 = hl.reshape(N_PAD, 2, DF).transpose(1, 0, 2)
  (p1,) = _segsum2(hl2, src, dst)
  p1c = p1.transpose(1, 2, 0, 3).reshape(NUM_SC, N_PAD, D_OUT)

  loss = pl.pallas_call(
      _loss_body,
      out_shape=jax.ShapeDtypeStruct((1, 1), f32),
      out_specs=pl.BlockSpec(memory_space=pltpu.SMEM),
  )(p1c, cnt3, hr, y_pad, m_pad)

  return loss.reshape(1)


# speedup vs baseline: 1.8056x; 1.0206x over previous
import jax
import jax.numpy as jnp
from jax import lax
from jax.experimental import pallas as pl
from jax.experimental.pallas import tpu as pltpu
from jax.experimental.pallas import tpu_sc as plsc

N = 10000
E = 320000
D_IN = 128
D_HID = 128
D_OUT = 64
DF = 32

NUM_SC = 2
NUM_TILES = 16
NW = NUM_SC * NUM_TILES
LANES = 16

CHUNK = 128
NBUF = 3
GLEAD = 1
SLAG = 2
EDGES_PER_TILE = -(-E // (NW * CHUNK * NBUF)) * CHUNK * NBUF
E_PAD = EDGES_PER_TILE * NW
N_PAD = 10112
ROWS_PER_TILE = N_PAD // NUM_TILES


def _make_segsum(npass, with_cnt):
  mesh = plsc.VectorSubcoreMesh(core_axis_name="c", subcore_axis_name="s")

  out_type = [jax.ShapeDtypeStruct((npass, NUM_SC, N_PAD, DF), jnp.float32)]
  if with_cnt:
    out_type.append(jax.ShapeDtypeStruct((NUM_SC, N_PAD), jnp.float32))

  nct = EDGES_PER_TILE // CHUNK

  scratch = [
      pltpu.VMEM((nct, CHUNK), jnp.int32),
      pltpu.VMEM((nct, CHUNK), jnp.int32),
      pltpu.VMEM((NBUF, CHUNK, DF), jnp.float32),
      pltpu.VMEM((ROWS_PER_TILE, DF), jnp.float32),
      pltpu.VMEM_SHARED((N_PAD, DF), jnp.float32),
      pltpu.VMEM_SHARED((N_PAD, DF), jnp.float32),
      [pltpu.SemaphoreType.DMA] * NBUF,
      [pltpu.SemaphoreType.DMA] * NBUF,
  ]
  if with_cnt:
    scratch += [
        pltpu.VMEM((CHUNK,), jnp.float32),
        pltpu.VMEM((ROWS_PER_TILE,), jnp.float32),
        pltpu.VMEM_SHARED((N_PAD,), jnp.float32),
        pltpu.SemaphoreType.DMA,
    ]

  def body(table, src, dst, *refs):
    if with_cnt:
      (out, cnt_out, srcv, dstv, rows, zbuf, stab, acc, gsem, ssem,
       ones, zbuf1, cntacc, csem) = refs
    else:
      out, srcv, dstv, rows, zbuf, stab, acc, gsem, ssem = refs
    cid = lax.axis_index("c")
    sid = lax.axis_index("s")
    wid = sid * NUM_SC + cid
    tile_row0 = wid * nct
    row0 = sid * ROWS_PER_TILE

    def fire_gather(b, i):
      pltpu.async_copy(stab.at[srcv.at[i]], rows.at[b], gsem[b])

    def drain_gather(b, i):
      pltpu.make_async_copy(stab.at[srcv.at[i]], rows.at[b], gsem[b]).wait()

    def fire_scatter(b, i, j):
      pltpu.async_copy(rows.at[b], acc.at[dstv.at[i]], ssem[b], add=True)
      if with_cnt:
        @pl.when(j == 0)
        def _():
          pltpu.async_copy(ones, cntacc.at[dstv.at[i]], csem, add=True)

    def drain_scatter(b, i):
      pltpu.make_async_copy(rows.at[b], acc.at[dstv.at[i]], ssem[b]).wait()

    zvec = jnp.zeros((LANES,), jnp.float32)

    def zrow(r, _):
      for j in range(DF // LANES):
        zbuf[r, pl.ds(j * LANES, LANES)] = zvec
      return 0
    lax.fori_loop(0, ROWS_PER_TILE, zrow, 0)
    if with_cnt:
      def zrow1(r, _):
        zbuf1[pl.ds(r * LANES, LANES)] = zvec
        return 0
      lax.fori_loop(0, ROWS_PER_TILE // LANES, zrow1, 0)
      onev = jnp.ones((LANES,), jnp.float32)
      for j in range(CHUNK // LANES):
        ones[pl.ds(j * LANES, LANES)] = onev
    pltpu.sync_copy(src.at[pl.ds(tile_row0, nct)], srcv)
    pltpu.sync_copy(dst.at[pl.ds(tile_row0, nct)], dstv)

    def one_pass(j, _):
      pltpu.sync_copy(table.at[j, pl.ds(row0, ROWS_PER_TILE)],
                      stab.at[pl.ds(row0, ROWS_PER_TILE)])
      pltpu.sync_copy(zbuf, acc.at[pl.ds(row0, ROWS_PER_TILE)])
      if with_cnt:
        @pl.when(j == 0)
        def _():
          pltpu.sync_copy(zbuf1, cntacc.at[pl.ds(row0, ROWS_PER_TILE)])

      plsc.subcore_barrier()

      for v in range(GLEAD):
        fire_gather(v % NBUF, v)

      def outer(g, _):
        for b in range(NBUF):
          v = g * NBUF + b

          @pl.when(v >= SLAG)
          def _():
            drain_scatter((b - SLAG) % NBUF, v - SLAG)

          @pl.when(v + GLEAD < nct)
          def _():
            fire_gather((b + GLEAD) % NBUF, v + GLEAD)
          drain_gather(b, v)
          fire_scatter(b, v, j)
        return 0
      lax.fori_loop(0, nct // NBUF, outer, 0)

      for v in range(nct - SLAG, nct):
        drain_scatter(v % NBUF, v)
      if with_cnt:
        @pl.when(j == 0)
        def _():
          def cdrain(i, _):
            pltpu.make_async_copy(ones, cntacc.at[dstv.at[i]], csem).wait()
            return 0
          lax.fori_loop(0, nct, cdrain, 0)

      plsc.subcore_barrier()

      pltpu.sync_copy(acc.at[pl.ds(row0, ROWS_PER_TILE)],
                      out.at[j, cid, pl.ds(row0, ROWS_PER_TILE)])
      if with_cnt:
        @pl.when(j == 0)
        def _():
          pltpu.sync_copy(cntacc.at[pl.ds(row0, ROWS_PER_TILE)],
                          cnt_out.at[cid, pl.ds(row0, ROWS_PER_TILE)])
      return 0

    lax.fori_loop(0, npass, one_pass, 0)

  return pl.kernel(
      body, out_type=out_type, mesh=mesh, scratch_types=scratch,
      compiler_params=pltpu.CompilerParams(use_tc_tiling_on_sc=False,
                                           needs_layout_passes=False),
      name=f"segsum{npass}_cnt{int(with_cnt)}")


_segsum4_cnt = _make_segsum(4, with_cnt=True)
_segsum2 = _make_segsum(2, with_cnt=False)


def _mid_body(p_ref, cnt_ref, x_ref, wl0_ref, wr0_ref, bias0_ref,
              wl1_ref, wr1_ref, bias1_ref, hl_ref, hr_ref):
  s = p_ref[0] + p_ref[1]
  cnt = jnp.maximum(cnt_ref[0] + cnt_ref[1], 1.0)
  aggr = s / cnt
  x = x_ref[...]
  lin = (lax.dot_general(aggr, wl0_ref[...], (((1,), (1,)), ((), ())),
                         preferred_element_type=jnp.float32)
         + lax.dot_general(x, wr0_ref[...], (((1,), (1,)), ((), ())),
                           preferred_element_type=jnp.float32))
  h = jnp.maximum(lin + bias0_ref[...], 0.0)
  hl_ref[...] = lax.dot_general(h, wl1_ref[...], (((1,), (1,)), ((), ())),
                                preferred_element_type=jnp.float32)
  hr_ref[...] = (lax.dot_general(h, wr1_ref[...], (((1,), (1,)), ((), ())),
                                 preferred_element_type=jnp.float32)
                 + bias1_ref[...])


def _loss_body(p_ref, cnt_ref, hr_ref, y_ref, m_ref, out_ref):
  s = p_ref[0] + p_ref[1]
  cnt = jnp.maximum(cnt_ref[0] + cnt_ref[1], 1.0)
  logits = s / cnt + hr_ref[...]
  mx = jnp.max(logits, axis=1, keepdims=True)
  lse = mx + jnp.log(jnp.sum(jnp.exp(logits - mx), axis=1, keepdims=True))
  logp = logits - lse
  cols = lax.broadcasted_iota(jnp.int32, (N_PAD, D_OUT), 1)
  onehot = cols == y_ref[...]
  nll = -jnp.sum(jnp.where(onehot, logp, 0.0), axis=1, keepdims=True)
  m = m_ref[...]
  num = jnp.sum(nll * m)
  den = jnp.maximum(jnp.sum(m), 1.0)
  out_ref[0, 0] = num / den


def kernel(x_chunks, adj_chunks, y_chunks, train_mask_chunks,
           W_l0, b_l0, W_r0, b_r0, W_l1, b_l1, W_r1, b_r1):
  f32 = jnp.float32
  x_pad = jnp.zeros((N_PAD, D_IN), f32).at[:N].set(x_chunks)
  pad_e = E_PAD - E
  src = jnp.concatenate([adj_chunks[0], jnp.full((pad_e,), N, jnp.int32)])
  src = src.reshape(E_PAD // CHUNK, CHUNK)
  dst = jnp.concatenate([adj_chunks[1], jnp.full((pad_e,), N, jnp.int32)])
  dst = dst.reshape(E_PAD // CHUNK, CHUNK)
  y_pad = jnp.zeros((N_PAD, 1), jnp.int32).at[:N, 0].set(y_chunks)
  m_pad = jnp.zeros((N_PAD, 1), f32).at[:N, 0].set(
      train_mask_chunks.astype(f32))
  bias0 = (b_l0 + b_r0)[None, :]
  bias1 = (b_l1 + b_r1)[None, :]

  x4 = x_pad.reshape(N_PAD, 4, DF).transpose(1, 0, 2)

  p0, cnt = _segsum4_cnt(x4, src, dst)
  p0c = p0.transpose(1, 2, 0, 3).reshape(NUM_SC, N_PAD, D_IN)
  cnt3 = cnt[:, :, None]

  hl, hr = pl.pallas_call(
      _mid_body,
      out_shape=[jax.ShapeDtypeStruct((N_PAD, D_OUT), f32),
                 jax.ShapeDtypeStruct((N_PAD, D_OUT), f32)],
  )(p0c, cnt3, x_pad, W_l0, W_r0, bias0, W_l1, W_r1, bias1)

  hl2 = hl.reshape(N_PAD, 2, DF).transpose(1, 0, 2)
  (p1,) = _segsum2(hl2, src, dst)
  p1c = p1.transpose(1, 2, 0, 3).reshape(NUM_SC, N_PAD, D_OUT)

  loss = pl.pallas_call(
      _loss_body,
      out_shape=jax.ShapeDtypeStruct((1, 1), f32),
      out_specs=pl.BlockSpec(memory_space=pltpu.SMEM),
  )(p1c, cnt3, hr, y_pad, m_pad)

  return loss.reshape(1)
